# Initial kernel scaffold; baseline (speedup 1.0000x reference)
import jax
import jax.numpy as jnp
from jax.experimental import pallas as pl


def kernel(x, d1, d0, dm1, mask, fe_W, fe_b, fm1_W, fm1_b, fu1_W, fu1_b, fmm1_W, fmm1_b, fum1_W, fum1_b, fm0_W, fm0_b, fu0_W, fu0_b):
    raise NotImplementedError("write your pallas kernel here")


# speedup vs baseline: 3.4939x; 3.4939x over previous
import functools

import jax
import jax.numpy as jnp
from jax import lax
from jax.experimental import pallas as pl
from jax.experimental.pallas import tpu as pltpu
from jax.experimental.pallas import tpu_sc as plsc

F = 12
GP = 16
NC, NS = 2, 16
NW = NC * NS
CH = 2048


def _front_body(x_ref, fewt_ref, feb_ref, p1_ref, p2_ref, p3_ref,
                h_ref, g1_ref, g2_ref, g3_ref):
    h = jnp.dot(x_ref[...], fewt_ref[...], preferred_element_type=jnp.float32)
    h = jnp.maximum(h + feb_ref[...], 0.0)
    h_ref[...] = h
    g1_ref[...] = jnp.dot(h, p1_ref[...], preferred_element_type=jnp.float32)
    g2_ref[...] = jnp.dot(h, p2_ref[...], preferred_element_type=jnp.float32)
    g3_ref[...] = jnp.dot(h, p3_ref[...], preferred_element_type=jnp.float32)


def _front(xs, fe_Wt, fe_b2, p1, p2, p3, interpret=False):
    N = xs.shape[0]
    R = 2000
    assert N % R == 0
    grid = (N // R,)
    full = lambda a: pl.BlockSpec(a.shape, lambda i: (0,) * a.ndim)
    return pl.pallas_call(
        _front_body,
        grid=grid,
        in_specs=[
            pl.BlockSpec((R, xs.shape[1]), lambda i: (i, 0)),
            full(fe_Wt), full(fe_b2), full(p1), full(p2), full(p3),
        ],
        out_specs=[
            pl.BlockSpec((R, F), lambda i: (i, 0)),
            pl.BlockSpec((R, GP), lambda i: (i, 0)),
            pl.BlockSpec((R, GP), lambda i: (i, 0)),
            pl.BlockSpec((R, GP), lambda i: (i, 0)),
        ],
        out_shape=[
            jax.ShapeDtypeStruct((N, F), jnp.float32),
            jax.ShapeDtypeStruct((N, GP), jnp.float32),
            jax.ShapeDtypeStruct((N, GP), jnp.float32),
            jax.ShapeDtypeStruct((N, GP), jnp.float32),
        ],
        interpret=interpret,
    )(xs, fe_Wt, fe_b2, p1, p2, p3)


def _sc_gather(g1, g2, g3, i1, i2, i3, totp):
    cpw = totp // NW // CH
    mesh = plsc.VectorSubcoreMesh(core_axis_name="c", subcore_axis_name="s")

    @functools.partial(
        pl.kernel,
        out_type=[jax.ShapeDtypeStruct((totp, GP), jnp.float32)] * 3,
        mesh=mesh,
        scratch_types=[
            pltpu.VMEM((CH,), jnp.int32),
            pltpu.VMEM((CH, GP), jnp.float32),
            pltpu.SemaphoreType.DMA,
        ],
        compiler_params=pltpu.CompilerParams(use_tc_tiling_on_sc=False),
    )
    def k(t1, t2, t3, x1, x2, x3, o1, o2, o3, idx_v, rows_v, sem):
        wid = lax.axis_index("s") * NC + lax.axis_index("c")
        for tab, ix, out in ((t1, x1, o1), (t2, x2, o2), (t3, x3, o3)):
            def body(c, carry, tab=tab, ix=ix, out=out):
                r0 = wid * (cpw * CH) + c * CH
                pltpu.sync_copy(ix.at[pl.ds(r0, CH)], idx_v)
                pltpu.async_copy(tab.at[idx_v], rows_v, sem).wait()
                pltpu.sync_copy(rows_v, out.at[pl.ds(r0, CH)])
                return carry

            lax.fori_loop(0, cpw, body, 0)

    return k(g1, g2, g3, i1, i2, i3)


def _back_body(rep_ref, nodes_ref, fold_ref, h_ref, d1_ref, d2_ref, d3_ref,
               ga1_ref, ga2_ref, ga3_ref,
               wdb1_ref, bmb1_ref, wuht1_ref, wumt1_ref, bu1_ref,
               wdb2_ref, bmb2_ref, wuht2_ref, wumt2_ref, bu2_ref,
               wdb3_ref, bmb3_ref, wuht3_ref, wumt3_ref, bu3_ref,
               o1_ref, o2_ref, o3_ref):
    h = h_ref[...]
    R = h.shape[0]
    rep = rep_ref[...]
    nodes = nodes_ref[...]
    fold = fold_ref[...]
    branches = (
        (d1_ref, ga1_ref, wdb1_ref, bmb1_ref, wuht1_ref, wumt1_ref, bu1_ref, o1_ref),
        (d2_ref, ga2_ref, wdb2_ref, bmb2_ref, wuht2_ref, wumt2_ref, bu2_ref, o2_ref),
        (d3_ref, ga3_ref, wdb3_ref, bmb3_ref, wuht3_ref, wumt3_ref, bu3_ref, o3_ref),
    )
    for d_ref, ga_ref, wdb_ref, bmb_ref, wuht_ref, wumt_ref, bu_ref, o_ref in branches:
        dp = d_ref[...]
        dpr = jnp.dot(dp, rep, preferred_element_type=jnp.float32)
        diff = dpr - nodes
        ep = jnp.exp(-(diff * diff) / 2.0 / jnp.float32(0.015) ** 2)
        tp = jnp.dot(ep, wdb_ref[...], preferred_element_type=jnp.float32) + bmb_ref[...]
        z = jnp.maximum(ga_ref[...] + tp, 0.0)
        z2 = z.reshape(R, -1, 128).sum(axis=1)
        s = jnp.dot(z2, fold, preferred_element_type=jnp.float32)
        u = (jnp.dot(h, wuht_ref[...], preferred_element_type=jnp.float32)
             + jnp.dot(s, wumt_ref[...], preferred_element_type=jnp.float32)
             + bu_ref[...])
        o_ref[...] = jax.nn.sigmoid(u)


def _back(h, dp1, dp2, dp3, ga1, ga2, ga3, wparams, interpret=False):
    N = h.shape[0]
    RP = dp1.shape[0] // N
    R = 1000
    assert N % R == 0
    grid = (N // R,)
    full = lambda a: pl.BlockSpec(a.shape, lambda i: (0,) * a.ndim)
    node_spec = pl.BlockSpec((R, F), lambda i: (i, 0))
    dist_spec = pl.BlockSpec((R * RP, 8), lambda i: (i, 0))
    ga_spec = pl.BlockSpec((R * RP, 128), lambda i: (i, 0))

    nodes = jnp.linspace(0.0, 0.3, F, dtype=jnp.float32)
    nodes_t = jnp.tile(jnp.pad(nodes, (0, GP - F)), 8).reshape(1, 128)
    rep = jnp.kron(jnp.eye(8, dtype=jnp.float32),
                   jnp.ones((1, GP), jnp.float32))
    fold = jnp.tile(jnp.eye(GP, dtype=jnp.float32)[:, :F], (8, 1))

    in_specs = [full(rep), full(nodes_t), full(fold), node_spec,
                dist_spec, dist_spec, dist_spec, ga_spec, ga_spec, ga_spec]
    in_specs += [full(w) for w in wparams]
    return pl.pallas_call(
        _back_body,
        grid=grid,
        in_specs=in_specs,
        out_specs=[node_spec] * 3,
        out_shape=[jax.ShapeDtypeStruct((N, F), jnp.float32)] * 3,
        compiler_params=pltpu.CompilerParams(vmem_limit_bytes=100 * 2**20),
        interpret=interpret,
    )(rep, nodes_t, fold, h, dp1, dp2, dp3, ga1, ga2, ga3, *wparams)


def kernel(x, d1, d0, dm1, mask, fe_W, fe_b, fm1_W, fm1_b, fu1_W, fu1_b,
           fmm1_W, fmm1_b, fum1_W, fum1_b, fm0_W, fm0_b, fu0_W, fu0_b):
    B, N, F_in = x.shape
    K = d1.shape[2]
    tot = N * K
    totp = ((tot + NW * CH - 1) // (NW * CH)) * (NW * CH)

    xs = x.reshape(N, F_in)

    def split_d(d):
        idx = d[0, :, :, 0].astype(jnp.int32).reshape(tot)
        idx = jnp.pad(idx, (0, totp - tot))
        return idx, d[0, :, :, 1].reshape(tot // 8, 8)

    i1, dp1 = split_d(d1)
    i0, dp0 = split_d(d0)
    im1, dpm1 = split_d(dm1)

    def prep(Wm, bm, Wu, bu):
        p = jnp.zeros((F, GP), jnp.float32).at[:, :F].set(Wm[:, :F].T)
        wdt16 = jnp.zeros((GP, GP), jnp.float32).at[:F, :F].set(Wm[:, F:].T)
        wdb = jnp.kron(jnp.eye(8, dtype=jnp.float32), wdt16)
        bmb = jnp.tile(jnp.pad(bm, (0, GP - F)), 8).reshape(1, 128)
        return (p, wdb, bmb, Wu[:, :F].T, Wu[:, F:].T, bu.reshape(1, F))

    p1, wdb1, bmb1, wuht1, wumt1, bu1 = prep(fm1_W, fm1_b, fu1_W, fu1_b)
    p0, wdb0, bmb0, wuht0, wumt0, bu0 = prep(fm0_W, fm0_b, fu0_W, fu0_b)
    pm1, wdbm1, bmbm1, wuhtm1, wumtm1, bum1 = prep(fmm1_W, fmm1_b, fum1_W, fum1_b)

    h, g1, g0, gm1 = _front(xs, fe_W.T, fe_b.reshape(1, F), p1, p0, pm1)

    ga1, ga0, gam1 = _sc_gather(g1, g0, gm1, i1, i0, im1, totp)
    ga1 = ga1.reshape(totp // 8, 128)
    ga0 = ga0.reshape(totp // 8, 128)
    gam1 = gam1.reshape(totp // 8, 128)

    wparams = (wdb1, bmb1, wuht1, wumt1, bu1,
               wdb0, bmb0, wuht0, wumt0, bu0,
               wdbm1, bmbm1, wuhtm1, wumtm1, bum1)
    o1, o0, om1 = _back(h, dp1, dp0, dpm1, ga1, ga0, gam1, wparams)

    return (o1.reshape(B, N, F), o0.reshape(B, N, F), om1.reshape(B, N, F))


# speedup vs baseline: 3.8286x; 1.0958x over previous
import functools

import jax
import jax.numpy as jnp
from jax import lax
from jax.experimental import pallas as pl
from jax.experimental.pallas import tpu as pltpu
from jax.experimental.pallas import tpu_sc as plsc

F = 12
GP = 16
NC, NS = 2, 16
NW = NC * NS
CH = 2048


def _front_body(x_ref, fewt_ref, feb_ref, p1_ref, p2_ref, p3_ref,
                h_ref, g1_ref, g2_ref, g3_ref):
    h = jnp.dot(x_ref[...], fewt_ref[...], preferred_element_type=jnp.float32)
    h = jnp.maximum(h + feb_ref[...], 0.0)
    h_ref[...] = h
    g1_ref[...] = jnp.dot(h, p1_ref[...], preferred_element_type=jnp.float32)
    g2_ref[...] = jnp.dot(h, p2_ref[...], preferred_element_type=jnp.float32)
    g3_ref[...] = jnp.dot(h, p3_ref[...], preferred_element_type=jnp.float32)


def _front(xs, fe_Wt, fe_b2, p1, p2, p3, interpret=False):
    N = xs.shape[0]
    R = 2000
    assert N % R == 0
    grid = (N // R,)
    full = lambda a: pl.BlockSpec(a.shape, lambda i: (0,) * a.ndim)
    return pl.pallas_call(
        _front_body,
        grid=grid,
        in_specs=[
            pl.BlockSpec((R, xs.shape[1]), lambda i: (i, 0)),
            full(fe_Wt), full(fe_b2), full(p1), full(p2), full(p3),
        ],
        out_specs=[
            pl.BlockSpec((R, F), lambda i: (i, 0)),
            pl.BlockSpec((R, GP), lambda i: (i, 0)),
            pl.BlockSpec((R, GP), lambda i: (i, 0)),
            pl.BlockSpec((R, GP), lambda i: (i, 0)),
        ],
        out_shape=[
            jax.ShapeDtypeStruct((N, F), jnp.float32),
            jax.ShapeDtypeStruct((N, GP), jnp.float32),
            jax.ShapeDtypeStruct((N, GP), jnp.float32),
            jax.ShapeDtypeStruct((N, GP), jnp.float32),
        ],
        interpret=interpret,
    )(xs, fe_Wt, fe_b2, p1, p2, p3)


def _sc_gather(g1, g2, g3, i1, i2, i3, totp):
    cpw = totp // NW // CH
    NBUF = 3
    mesh = plsc.VectorSubcoreMesh(core_axis_name="c", subcore_axis_name="s")

    @functools.partial(
        pl.kernel,
        out_type=[jax.ShapeDtypeStruct((totp, GP), jnp.float32)] * 3,
        mesh=mesh,
        scratch_types=(
            [pltpu.VMEM((CH,), jnp.int32)] * NBUF
            + [pltpu.VMEM((CH, GP), jnp.float32)] * NBUF
            + [pltpu.SemaphoreType.DMA] * (3 * NBUF)
        ),
        compiler_params=pltpu.CompilerParams(use_tc_tiling_on_sc=False),
    )
    def k(t1, t2, t3, x1, x2, x3, o1, o2, o3, *scr):
        idx_v = scr[:NBUF]
        rows_v = scr[NBUF:2 * NBUF]
        xsem = scr[2 * NBUF:3 * NBUF]
        gsem = scr[3 * NBUF:4 * NBUF]
        ssem = scr[4 * NBUF:5 * NBUF]
        wid = lax.axis_index("s") * NC + lax.axis_index("c")
        base = wid * (cpw * CH)

        chunks = [(tab, ix, out, c)
                  for tab, ix, out in ((t1, x1, o1), (t2, x2, o2), (t3, x3, o3))
                  for c in range(cpw)]
        n = len(chunks)

        def issue_idx(j):
            _, ix, _, c = chunks[j]
            return pltpu.async_copy(ix.at[pl.ds(base + c * CH, CH)],
                                    idx_v[j % NBUF], xsem[j % NBUF])

        idx_h = {j: issue_idx(j) for j in range(min(NBUF, n))}
        g_h, s_h = {}, {}
        for j in range(n + 1):
            b = j % NBUF
            if j < n:
                tab = chunks[j][0]
                idx_h[j].wait()
                if j >= NBUF:
                    s_h[j - NBUF].wait()
                g_h[j] = pltpu.async_copy(tab.at[idx_v[b]], rows_v[b], gsem[b])
            if j >= 1:
                bp = (j - 1) % NBUF
                _, _, out, c = chunks[j - 1]
                g_h[j - 1].wait()
                s_h[j - 1] = pltpu.async_copy(
                    rows_v[bp], out.at[pl.ds(base + c * CH, CH)], ssem[bp])
                if j - 1 + NBUF < n:
                    idx_h[j - 1 + NBUF] = issue_idx(j - 1 + NBUF)
        for j in range(max(0, n - NBUF), n):
            s_h[j].wait()

    return k(g1, g2, g3, i1, i2, i3)


def _back_body(rep_ref, nodes_ref, fold_ref, h_ref, d1_ref, d2_ref, d3_ref,
               ga1_ref, ga2_ref, ga3_ref,
               wdb1_ref, bmb1_ref, wuht1_ref, wumt1_ref, bu1_ref,
               wdb2_ref, bmb2_ref, wuht2_ref, wumt2_ref, bu2_ref,
               wdb3_ref, bmb3_ref, wuht3_ref, wumt3_ref, bu3_ref,
               o1_ref, o2_ref, o3_ref):
    h = h_ref[...]
    R = h.shape[0]
    rep = rep_ref[...]
    nodes = nodes_ref[...]
    fold = fold_ref[...]
    branches = (
        (d1_ref, ga1_ref, wdb1_ref, bmb1_ref, wuht1_ref, wumt1_ref, bu1_ref, o1_ref),
        (d2_ref, ga2_ref, wdb2_ref, bmb2_ref, wuht2_ref, wumt2_ref, bu2_ref, o2_ref),
        (d3_ref, ga3_ref, wdb3_ref, bmb3_ref, wuht3_ref, wumt3_ref, bu3_ref, o3_ref),
    )
    for d_ref, ga_ref, wdb_ref, bmb_ref, wuht_ref, wumt_ref, bu_ref, o_ref in branches:
        dp = d_ref[...]
        dpr = jnp.dot(dp, rep, preferred_element_type=jnp.float32)
        diff = dpr - nodes
        ep = jnp.exp(-(diff * diff) / 2.0 / jnp.float32(0.015) ** 2)
        tp = jnp.dot(ep, wdb_ref[...], preferred_element_type=jnp.float32) + bmb_ref[...]
        z = jnp.maximum(ga_ref[...] + tp, 0.0)
        z2 = z.reshape(R, -1, 128).sum(axis=1)
        s = jnp.dot(z2, fold, preferred_element_type=jnp.float32)
        u = (jnp.dot(h, wuht_ref[...], preferred_element_type=jnp.float32)
             + jnp.dot(s, wumt_ref[...], preferred_element_type=jnp.float32)
             + bu_ref[...])
        o_ref[...] = jax.nn.sigmoid(u)


def _back(h, dp1, dp2, dp3, ga1, ga2, ga3, wparams, interpret=False):
    N = h.shape[0]
    RP = dp1.shape[0] // N
    R = 1000
    assert N % R == 0
    grid = (N // R,)
    full = lambda a: pl.BlockSpec(a.shape, lambda i: (0,) * a.ndim)
    node_spec = pl.BlockSpec((R, F), lambda i: (i, 0))
    dist_spec = pl.BlockSpec((R * RP, 8), lambda i: (i, 0))
    ga_spec = pl.BlockSpec((R * RP, 128), lambda i: (i, 0))

    nodes = jnp.linspace(0.0, 0.3, F, dtype=jnp.float32)
    nodes_t = jnp.tile(jnp.pad(nodes, (0, GP - F)), 8).reshape(1, 128)
    rep = jnp.kron(jnp.eye(8, dtype=jnp.float32),
                   jnp.ones((1, GP), jnp.float32))
    fold = jnp.tile(jnp.eye(GP, dtype=jnp.float32)[:, :F], (8, 1))

    in_specs = [full(rep), full(nodes_t), full(fold), node_spec,
                dist_spec, dist_spec, dist_spec, ga_spec, ga_spec, ga_spec]
    in_specs += [full(w) for w in wparams]
    return pl.pallas_call(
        _back_body,
        grid=grid,
        in_specs=in_specs,
        out_specs=[node_spec] * 3,
        out_shape=[jax.ShapeDtypeStruct((N, F), jnp.float32)] * 3,
        compiler_params=pltpu.CompilerParams(vmem_limit_bytes=100 * 2**20),
        interpret=interpret,
    )(rep, nodes_t, fold, h, dp1, dp2, dp3, ga1, ga2, ga3, *wparams)


def kernel(x, d1, d0, dm1, mask, fe_W, fe_b, fm1_W, fm1_b, fu1_W, fu1_b,
           fmm1_W, fmm1_b, fum1_W, fum1_b, fm0_W, fm0_b, fu0_W, fu0_b):
    B, N, F_in = x.shape
    K = d1.shape[2]
    tot = N * K
    totp = ((tot + NW * CH - 1) // (NW * CH)) * (NW * CH)

    xs = x.reshape(N, F_in)

    def split_d(d):
        idx = d[0, :, :, 0].astype(jnp.int32).reshape(tot)
        idx = jnp.pad(idx, (0, totp - tot))
        return idx, d[0, :, :, 1].reshape(tot // 8, 8)

    i1, dp1 = split_d(d1)
    i0, dp0 = split_d(d0)
    im1, dpm1 = split_d(dm1)

    def prep(Wm, bm, Wu, bu):
        p = jnp.zeros((F, GP), jnp.float32).at[:, :F].set(Wm[:, :F].T)
        wdt16 = jnp.zeros((GP, GP), jnp.float32).at[:F, :F].set(Wm[:, F:].T)
        wdb = jnp.kron(jnp.eye(8, dtype=jnp.float32), wdt16)
        bmb = jnp.tile(jnp.pad(bm, (0, GP - F)), 8).reshape(1, 128)
        return (p, wdb, bmb, Wu[:, :F].T, Wu[:, F:].T, bu.reshape(1, F))

    p1, wdb1, bmb1, wuht1, wumt1, bu1 = prep(fm1_W, fm1_b, fu1_W, fu1_b)
    p0, wdb0, bmb0, wuht0, wumt0, bu0 = prep(fm0_W, fm0_b, fu0_W, fu0_b)
    pm1, wdbm1, bmbm1, wuhtm1, wumtm1, bum1 = prep(fmm1_W, fmm1_b, fum1_W, fum1_b)

    h, g1, g0, gm1 = _front(xs, fe_W.T, fe_b.reshape(1, F), p1, p0, pm1)

    ga1, ga0, gam1 = _sc_gather(g1, g0, gm1, i1, i0, im1, totp)
    ga1 = ga1.reshape(totp // 8, 128)
    ga0 = ga0.reshape(totp // 8, 128)
    gam1 = gam1.reshape(totp // 8, 128)

    wparams = (wdb1, bmb1, wuht1, wumt1, bu1,
               wdb0, bmb0, wuht0, wumt0, bu0,
               wdbm1, bmbm1, wuhtm1, wumtm1, bum1)
    o1, o0, om1 = _back(h, dp1, dp0, dpm1, ga1, ga0, gam1, wparams)

    return (o1.reshape(B, N, F), o0.reshape(B, N, F), om1.reshape(B, N, F))


# speedup vs baseline: 4.0185x; 1.0496x over previous
import functools

import jax
import jax.numpy as jnp
from jax import lax
from jax.experimental import pallas as pl
from jax.experimental.pallas import tpu as pltpu
from jax.experimental.pallas import tpu_sc as plsc

F = 12
GP = 16
NC, NS = 2, 16
NW = NC * NS
CH = 2048


def _front_body(x_ref, fewt_ref, feb_ref, p1_ref, p2_ref, p3_ref,
                h_ref, g1_ref, g2_ref, g3_ref):
    h = jnp.dot(x_ref[...], fewt_ref[...], preferred_element_type=jnp.float32)
    h = jnp.maximum(h + feb_ref[...], 0.0)
    h_ref[...] = h
    g1_ref[...] = jnp.dot(h, p1_ref[...], preferred_element_type=jnp.float32).astype(jnp.bfloat16)
    g2_ref[...] = jnp.dot(h, p2_ref[...], preferred_element_type=jnp.float32).astype(jnp.bfloat16)
    g3_ref[...] = jnp.dot(h, p3_ref[...], preferred_element_type=jnp.float32).astype(jnp.bfloat16)


def _front(xs, fe_Wt, fe_b2, p1, p2, p3, interpret=False):
    N = xs.shape[0]
    R = 2000
    assert N % R == 0
    grid = (N // R,)
    full = lambda a: pl.BlockSpec(a.shape, lambda i: (0,) * a.ndim)
    return pl.pallas_call(
        _front_body,
        grid=grid,
        in_specs=[
            pl.BlockSpec((R, xs.shape[1]), lambda i: (i, 0)),
            full(fe_Wt), full(fe_b2), full(p1), full(p2), full(p3),
        ],
        out_specs=[
            pl.BlockSpec((R, F), lambda i: (i, 0)),
            pl.BlockSpec((R, GP), lambda i: (i, 0)),
            pl.BlockSpec((R, GP), lambda i: (i, 0)),
            pl.BlockSpec((R, GP), lambda i: (i, 0)),
        ],
        out_shape=[
            jax.ShapeDtypeStruct((N, F), jnp.float32),
            jax.ShapeDtypeStruct((N, GP), jnp.bfloat16),
            jax.ShapeDtypeStruct((N, GP), jnp.bfloat16),
            jax.ShapeDtypeStruct((N, GP), jnp.bfloat16),
        ],
        interpret=interpret,
    )(xs, fe_Wt, fe_b2, p1, p2, p3)


def _sc_gather(g1, g2, g3, i1, i2, i3, totp):
    cpw = totp // NW // CH
    NBUF = 3
    mesh = plsc.VectorSubcoreMesh(core_axis_name="c", subcore_axis_name="s")

    N = g1.shape[0]
    rps = N // NS
    assert rps * NS == N

    @functools.partial(
        pl.kernel,
        out_type=[jax.ShapeDtypeStruct((totp, GP), jnp.bfloat16)] * 3,
        mesh=mesh,
        scratch_types=(
            [pltpu.VMEM((CH,), jnp.int32)] * NBUF
            + [pltpu.VMEM((CH, GP), jnp.bfloat16)] * NBUF
            + [pltpu.VMEM_SHARED((N, GP), jnp.bfloat16)]
            + [pltpu.SemaphoreType.DMA] * (3 * NBUF)
        ),
        compiler_params=pltpu.CompilerParams(use_tc_tiling_on_sc=False),
    )
    def k(t1, t2, t3, x1, x2, x3, o1, o2, o3, *scr):
        idx_v = scr[:NBUF]
        rows_v = scr[NBUF:2 * NBUF]
        sp_tab = scr[2 * NBUF]
        xsem = scr[2 * NBUF + 1:3 * NBUF + 1]
        gsem = scr[3 * NBUF + 1:4 * NBUF + 1]
        ssem = scr[4 * NBUF + 1:5 * NBUF + 1]
        sid = lax.axis_index("s")
        wid = sid * NC + lax.axis_index("c")
        base = wid * (cpw * CH)
        s_pend = {}

        def run_branch(tab_hbm, ix, out):
            pltpu.sync_copy(tab_hbm.at[pl.ds(sid * rps, rps)],
                            sp_tab.at[pl.ds(sid * rps, rps)])
            plsc.subcore_barrier()

            def issue_idx(c):
                return pltpu.async_copy(ix.at[pl.ds(base + c * CH, CH)],
                                        idx_v[c % NBUF], xsem[c % NBUF])

            idx_h = {c: issue_idx(c) for c in range(min(NBUF, cpw))}
            g_h = {}
            for j in range(cpw + 1):
                b = j % NBUF
                if j < cpw:
                    idx_h[j].wait()
                    if b in s_pend:
                        s_pend.pop(b).wait()
                    g_h[j] = pltpu.async_copy(sp_tab.at[idx_v[b]], rows_v[b],
                                              gsem[b])
                if j >= 1:
                    bp = (j - 1) % NBUF
                    g_h[j - 1].wait()
                    s_pend[bp] = pltpu.async_copy(
                        rows_v[bp], out.at[pl.ds(base + (j - 1) * CH, CH)],
                        ssem[bp])
                    if j - 1 + NBUF < cpw:
                        idx_h[j - 1 + NBUF] = issue_idx(j - 1 + NBUF)
            plsc.subcore_barrier()

        for tab_hbm, ix, out in ((t1, x1, o1), (t2, x2, o2), (t3, x3, o3)):
            run_branch(tab_hbm, ix, out)
        for h in s_pend.values():
            h.wait()

    return k(g1, g2, g3, i1, i2, i3)


def _back_body(rep_ref, nodes_ref, fold_ref, h_ref, d1_ref, d2_ref, d3_ref,
               ga1_ref, ga2_ref, ga3_ref,
               wdb1_ref, bmb1_ref, wuht1_ref, wumt1_ref, bu1_ref,
               wdb2_ref, bmb2_ref, wuht2_ref, wumt2_ref, bu2_ref,
               wdb3_ref, bmb3_ref, wuht3_ref, wumt3_ref, bu3_ref,
               o1_ref, o2_ref, o3_ref):
    h = h_ref[...]
    R = h.shape[0]
    rep = rep_ref[...]
    nodes = nodes_ref[...]
    fold = fold_ref[...]
    branches = (
        (d1_ref, ga1_ref, wdb1_ref, bmb1_ref, wuht1_ref, wumt1_ref, bu1_ref, o1_ref),
        (d2_ref, ga2_ref, wdb2_ref, bmb2_ref, wuht2_ref, wumt2_ref, bu2_ref, o2_ref),
        (d3_ref, ga3_ref, wdb3_ref, bmb3_ref, wuht3_ref, wumt3_ref, bu3_ref, o3_ref),
    )
    for d_ref, ga_ref, wdb_ref, bmb_ref, wuht_ref, wumt_ref, bu_ref, o_ref in branches:
        dp = d_ref[...]
        dpr = jnp.dot(dp, rep, preferred_element_type=jnp.float32)
        diff = dpr - nodes
        ep = jnp.exp(-(diff * diff) / 2.0 / jnp.float32(0.015) ** 2)
        tp = jnp.dot(ep, wdb_ref[...], preferred_element_type=jnp.float32) + bmb_ref[...]
        z = jnp.maximum(ga_ref[...].astype(jnp.float32) + tp, 0.0)
        z2 = z.reshape(R, -1, 128).sum(axis=1)
        s = jnp.dot(z2, fold, preferred_element_type=jnp.float32)
        u = (jnp.dot(h, wuht_ref[...], preferred_element_type=jnp.float32)
             + jnp.dot(s, wumt_ref[...], preferred_element_type=jnp.float32)
             + bu_ref[...])
        o_ref[...] = jax.nn.sigmoid(u)


def _back(h, dp1, dp2, dp3, ga1, ga2, ga3, wparams, interpret=False):
    N = h.shape[0]
    RP = dp1.shape[0] // N
    R = 1000
    assert N % R == 0
    grid = (N // R,)
    full = lambda a: pl.BlockSpec(a.shape, lambda i: (0,) * a.ndim)
    node_spec = pl.BlockSpec((R, F), lambda i: (i, 0))
    dist_spec = pl.BlockSpec((R * RP, 8), lambda i: (i, 0))
    ga_spec = pl.BlockSpec((R * RP, 128), lambda i: (i, 0))

    nodes = jnp.linspace(0.0, 0.3, F, dtype=jnp.float32)
    nodes_t = jnp.tile(jnp.pad(nodes, (0, GP - F)), 8).reshape(1, 128)
    rep = jnp.kron(jnp.eye(8, dtype=jnp.float32),
                   jnp.ones((1, GP), jnp.float32))
    fold = jnp.tile(jnp.eye(GP, dtype=jnp.float32)[:, :F], (8, 1))

    in_specs = [full(rep), full(nodes_t), full(fold), node_spec,
                dist_spec, dist_spec, dist_spec, ga_spec, ga_spec, ga_spec]
    in_specs += [full(w) for w in wparams]
    return pl.pallas_call(
        _back_body,
        grid=grid,
        in_specs=in_specs,
        out_specs=[node_spec] * 3,
        out_shape=[jax.ShapeDtypeStruct((N, F), jnp.float32)] * 3,
        compiler_params=pltpu.CompilerParams(vmem_limit_bytes=100 * 2**20),
        interpret=interpret,
    )(rep, nodes_t, fold, h, dp1, dp2, dp3, ga1, ga2, ga3, *wparams)


def kernel(x, d1, d0, dm1, mask, fe_W, fe_b, fm1_W, fm1_b, fu1_W, fu1_b,
           fmm1_W, fmm1_b, fum1_W, fum1_b, fm0_W, fm0_b, fu0_W, fu0_b):
    B, N, F_in = x.shape
    K = d1.shape[2]
    tot = N * K
    totp = ((tot + NW * CH - 1) // (NW * CH)) * (NW * CH)

    xs = x.reshape(N, F_in)

    def split_d(d):
        idx = d[0, :, :, 0].astype(jnp.int32).reshape(tot)
        idx = jnp.pad(idx, (0, totp - tot))
        return idx, d[0, :, :, 1].reshape(tot // 8, 8)

    i1, dp1 = split_d(d1)
    i0, dp0 = split_d(d0)
    im1, dpm1 = split_d(dm1)

    def prep(Wm, bm, Wu, bu):
        p = jnp.zeros((F, GP), jnp.float32).at[:, :F].set(Wm[:, :F].T)
        wdt16 = jnp.zeros((GP, GP), jnp.float32).at[:F, :F].set(Wm[:, F:].T)
        wdb = jnp.kron(jnp.eye(8, dtype=jnp.float32), wdt16)
        bmb = jnp.tile(jnp.pad(bm, (0, GP - F)), 8).reshape(1, 128)
        return (p, wdb, bmb, Wu[:, :F].T, Wu[:, F:].T, bu.reshape(1, F))

    p1, wdb1, bmb1, wuht1, wumt1, bu1 = prep(fm1_W, fm1_b, fu1_W, fu1_b)
    p0, wdb0, bmb0, wuht0, wumt0, bu0 = prep(fm0_W, fm0_b, fu0_W, fu0_b)
    pm1, wdbm1, bmbm1, wuhtm1, wumtm1, bum1 = prep(fmm1_W, fmm1_b, fum1_W, fum1_b)

    h, g1, g0, gm1 = _front(xs, fe_W.T, fe_b.reshape(1, F), p1, p0, pm1)

    ga1, ga0, gam1 = _sc_gather(g1, g0, gm1, i1, i0, im1, totp)
    ga1 = ga1.reshape(totp // 8, 128)
    ga0 = ga0.reshape(totp // 8, 128)
    gam1 = gam1.reshape(totp // 8, 128)

    wparams = (wdb1, bmb1, wuht1, wumt1, bu1,
               wdb0, bmb0, wuht0, wumt0, bu0,
               wdbm1, bmbm1, wuhtm1, wumtm1, bum1)
    o1, o0, om1 = _back(h, dp1, dp0, dpm1, ga1, ga0, gam1, wparams)

    return (o1.reshape(B, N, F), o0.reshape(B, N, F), om1.reshape(B, N, F))


# speedup vs baseline: 4.0214x; 1.0007x over previous
import functools

import jax
import jax.numpy as jnp
from jax import lax
from jax.experimental import pallas as pl
from jax.experimental.pallas import tpu as pltpu
from jax.experimental.pallas import tpu_sc as plsc

F = 12
GP = 16
NC, NS = 2, 16
NW = NC * NS
CH = 2560


def _front_body(x_ref, fewt_ref, feb_ref, p1_ref, p2_ref, p3_ref,
                h_ref, g1_ref, g2_ref, g3_ref):
    h = jnp.dot(x_ref[...], fewt_ref[...], preferred_element_type=jnp.float32)
    h = jnp.maximum(h + feb_ref[...], 0.0)
    h_ref[...] = h
    g1_ref[...] = jnp.dot(h, p1_ref[...], preferred_element_type=jnp.float32).astype(jnp.bfloat16)
    g2_ref[...] = jnp.dot(h, p2_ref[...], preferred_element_type=jnp.float32).astype(jnp.bfloat16)
    g3_ref[...] = jnp.dot(h, p3_ref[...], preferred_element_type=jnp.float32).astype(jnp.bfloat16)


def _front(xs, fe_Wt, fe_b2, p1, p2, p3, interpret=False):
    N = xs.shape[0]
    R = 2000
    assert N % R == 0
    grid = (N // R,)
    full = lambda a: pl.BlockSpec(a.shape, lambda i: (0,) * a.ndim)
    return pl.pallas_call(
        _front_body,
        grid=grid,
        in_specs=[
            pl.BlockSpec((R, xs.shape[1]), lambda i: (i, 0)),
            full(fe_Wt), full(fe_b2), full(p1), full(p2), full(p3),
        ],
        out_specs=[
            pl.BlockSpec((R, F), lambda i: (i, 0)),
            pl.BlockSpec((R, GP), lambda i: (i, 0)),
            pl.BlockSpec((R, GP), lambda i: (i, 0)),
            pl.BlockSpec((R, GP), lambda i: (i, 0)),
        ],
        out_shape=[
            jax.ShapeDtypeStruct((N, F), jnp.float32),
            jax.ShapeDtypeStruct((N, GP), jnp.bfloat16),
            jax.ShapeDtypeStruct((N, GP), jnp.bfloat16),
            jax.ShapeDtypeStruct((N, GP), jnp.bfloat16),
        ],
        interpret=interpret,
    )(xs, fe_Wt, fe_b2, p1, p2, p3)


def _sc_gather(g1, g2, g3, i1, i2, i3, totp):
    cpw = totp // NW // CH
    NBUF = 4
    assert cpw % NBUF == 0
    mesh = plsc.VectorSubcoreMesh(core_axis_name="c", subcore_axis_name="s")

    N = g1.shape[0]
    rps = N // NS
    assert rps * NS == N

    @functools.partial(
        pl.kernel,
        out_type=[jax.ShapeDtypeStruct((totp, GP), jnp.bfloat16)] * 3,
        mesh=mesh,
        scratch_types=(
            [pltpu.VMEM((CH,), jnp.int32)] * NBUF
            + [pltpu.VMEM((CH, GP), jnp.bfloat16)] * NBUF
            + [pltpu.VMEM_SHARED((N, GP), jnp.bfloat16)]
            + [pltpu.SemaphoreType.DMA] * (3 * NBUF)
        ),
        compiler_params=pltpu.CompilerParams(use_tc_tiling_on_sc=False),
    )
    def k(t1, t2, t3, x1, x2, x3, o1, o2, o3, *scr):
        idx_v = scr[:NBUF]
        rows_v = scr[NBUF:2 * NBUF]
        sp_tab = scr[2 * NBUF]
        xsem = scr[2 * NBUF + 1:3 * NBUF + 1]
        gsem = scr[3 * NBUF + 1:4 * NBUF + 1]
        ssem = scr[4 * NBUF + 1:5 * NBUF + 1]
        sid = lax.axis_index("s")
        wid = sid * NC + lax.axis_index("c")
        base = wid * (cpw * CH)

        def run_branch(tab_hbm, ix, out):
            pltpu.sync_copy(tab_hbm.at[pl.ds(sid * rps, rps)],
                            sp_tab.at[pl.ds(sid * rps, rps)])
            plsc.subcore_barrier()

            def group(g, carry):
                c0 = base + g * (NBUF * CH)
                hx = [pltpu.async_copy(ix.at[pl.ds(c0 + b * CH, CH)],
                                       idx_v[b], xsem[b])
                      for b in range(NBUF)]
                hg = []
                for b in range(NBUF):
                    hx[b].wait()
                    hg.append(pltpu.async_copy(sp_tab.at[idx_v[b]], rows_v[b],
                                               gsem[b]))
                hs = []
                for b in range(NBUF):
                    hg[b].wait()
                    hs.append(pltpu.async_copy(
                        rows_v[b], out.at[pl.ds(c0 + b * CH, CH)], ssem[b]))
                for b in range(NBUF):
                    hs[b].wait()
                return carry

            lax.fori_loop(0, cpw // NBUF, group, 0)
            plsc.subcore_barrier()

        for tab_hbm, ix, out in ((t1, x1, o1), (t2, x2, o2), (t3, x3, o3)):
            run_branch(tab_hbm, ix, out)

    return k(g1, g2, g3, i1, i2, i3)


def _back_body(rep_ref, nodes_ref, fold_ref, h_ref, d1_ref, d2_ref, d3_ref,
               ga1_ref, ga2_ref, ga3_ref,
               wdb1_ref, bmb1_ref, wuht1_ref, wumt1_ref, bu1_ref,
               wdb2_ref, bmb2_ref, wuht2_ref, wumt2_ref, bu2_ref,
               wdb3_ref, bmb3_ref, wuht3_ref, wumt3_ref, bu3_ref,
               o1_ref, o2_ref, o3_ref):
    h = h_ref[...]
    R = h.shape[0]
    rep = rep_ref[...]
    nodes = nodes_ref[...]
    fold = fold_ref[...]
    branches = (
        (d1_ref, ga1_ref, wdb1_ref, bmb1_ref, wuht1_ref, wumt1_ref, bu1_ref, o1_ref),
        (d2_ref, ga2_ref, wdb2_ref, bmb2_ref, wuht2_ref, wumt2_ref, bu2_ref, o2_ref),
        (d3_ref, ga3_ref, wdb3_ref, bmb3_ref, wuht3_ref, wumt3_ref, bu3_ref, o3_ref),
    )
    for d_ref, ga_ref, wdb_ref, bmb_ref, wuht_ref, wumt_ref, bu_ref, o_ref in branches:
        dp = d_ref[...]
        dpr = jnp.dot(dp, rep, preferred_element_type=jnp.float32)
        diff = dpr - nodes
        ep = jnp.exp(-(diff * diff) / 2.0 / jnp.float32(0.015) ** 2)
        tp = jnp.dot(ep, wdb_ref[...], preferred_element_type=jnp.float32) + bmb_ref[...]
        z = jnp.maximum(ga_ref[...].astype(jnp.float32) + tp, 0.0)
        z2 = z.reshape(R, -1, 128).sum(axis=1)
        s = jnp.dot(z2, fold, preferred_element_type=jnp.float32)
        u = (jnp.dot(h, wuht_ref[...], preferred_element_type=jnp.float32)
             + jnp.dot(s, wumt_ref[...], preferred_element_type=jnp.float32)
             + bu_ref[...])
        o_ref[...] = jax.nn.sigmoid(u)


def _back(h, dp1, dp2, dp3, ga1, ga2, ga3, wparams, interpret=False):
    N = h.shape[0]
    RP = dp1.shape[0] // N
    R = 1000
    assert N % R == 0
    grid = (N // R,)
    full = lambda a: pl.BlockSpec(a.shape, lambda i: (0,) * a.ndim)
    node_spec = pl.BlockSpec((R, F), lambda i: (i, 0))
    dist_spec = pl.BlockSpec((R * RP, 8), lambda i: (i, 0))
    ga_spec = pl.BlockSpec((R * RP, 128), lambda i: (i, 0))

    nodes = jnp.linspace(0.0, 0.3, F, dtype=jnp.float32)
    nodes_t = jnp.tile(jnp.pad(nodes, (0, GP - F)), 8).reshape(1, 128)
    rep = jnp.kron(jnp.eye(8, dtype=jnp.float32),
                   jnp.ones((1, GP), jnp.float32))
    fold = jnp.tile(jnp.eye(GP, dtype=jnp.float32)[:, :F], (8, 1))

    in_specs = [full(rep), full(nodes_t), full(fold), node_spec,
                dist_spec, dist_spec, dist_spec, ga_spec, ga_spec, ga_spec]
    in_specs += [full(w) for w in wparams]
    return pl.pallas_call(
        _back_body,
        grid=grid,
        in_specs=in_specs,
        out_specs=[node_spec] * 3,
        out_shape=[jax.ShapeDtypeStruct((N, F), jnp.float32)] * 3,
        compiler_params=pltpu.CompilerParams(vmem_limit_bytes=100 * 2**20),
        interpret=interpret,
    )(rep, nodes_t, fold, h, dp1, dp2, dp3, ga1, ga2, ga3, *wparams)


def kernel(x, d1, d0, dm1, mask, fe_W, fe_b, fm1_W, fm1_b, fu1_W, fu1_b,
           fmm1_W, fmm1_b, fum1_W, fum1_b, fm0_W, fm0_b, fu0_W, fu0_b):
    B, N, F_in = x.shape
    K = d1.shape[2]
    tot = N * K
    totp = ((tot + NW * CH - 1) // (NW * CH)) * (NW * CH)

    xs = x.reshape(N, F_in)

    def split_d(d):
        idx = d[0, :, :, 0].astype(jnp.int32).reshape(tot)
        idx = jnp.pad(idx, (0, totp - tot))
        return idx, d[0, :, :, 1].reshape(tot // 8, 8)

    i1, dp1 = split_d(d1)
    i0, dp0 = split_d(d0)
    im1, dpm1 = split_d(dm1)

    def prep(Wm, bm, Wu, bu):
        p = jnp.zeros((F, GP), jnp.float32).at[:, :F].set(Wm[:, :F].T)
        wdt16 = jnp.zeros((GP, GP), jnp.float32).at[:F, :F].set(Wm[:, F:].T)
        wdb = jnp.kron(jnp.eye(8, dtype=jnp.float32), wdt16)
        bmb = jnp.tile(jnp.pad(bm, (0, GP - F)), 8).reshape(1, 128)
        return (p, wdb, bmb, Wu[:, :F].T, Wu[:, F:].T, bu.reshape(1, F))

    p1, wdb1, bmb1, wuht1, wumt1, bu1 = prep(fm1_W, fm1_b, fu1_W, fu1_b)
    p0, wdb0, bmb0, wuht0, wumt0, bu0 = prep(fm0_W, fm0_b, fu0_W, fu0_b)
    pm1, wdbm1, bmbm1, wuhtm1, wumtm1, bum1 = prep(fmm1_W, fmm1_b, fum1_W, fum1_b)

    h, g1, g0, gm1 = _front(xs, fe_W.T, fe_b.reshape(1, F), p1, p0, pm1)

    ga1, ga0, gam1 = _sc_gather(g1, g0, gm1, i1, i0, im1, totp)
    ga1 = ga1.reshape(totp // 8, 128)
    ga0 = ga0.reshape(totp // 8, 128)
    gam1 = gam1.reshape(totp // 8, 128)

    wparams = (wdb1, bmb1, wuht1, wumt1, bu1,
               wdb0, bmb0, wuht0, wumt0, bu0,
               wdbm1, bmbm1, wuhtm1, wumtm1, bum1)
    o1, o0, om1 = _back(h, dp1, dp0, dpm1, ga1, ga0, gam1, wparams)

    return (o1.reshape(B, N, F), o0.reshape(B, N, F), om1.reshape(B, N, F))
